# baseline (speedup 1.0000x reference)
import jax
import jax.numpy as jnp
from jax import lax
from jax.experimental import pallas as pl
from jax.experimental.pallas import tpu as pltpu
from jax.experimental.pallas import tpu_sc as plsc

B = 32
T = 2048
D = 128
C = 128
NC = 2
NS = 16
NW = NC * NS
NTC = T // C
NBG = NW // NTC
GB = B // NBG
LANES = 16
NBUF = 6
DEPTH = 5


def _emb_body(x_hbm, tok_hbm, pos_hbm, out_hbm,
              pos_v, idx_v, tok0, tok1, tok2, tok3, tok4, tok5,
              psem,
              gsem0, gsem1, gsem2, gsem3, gsem4, gsem5,
              osem0, osem1, osem2, osem3, osem4, osem5):
    wid = lax.axis_index("s") * NC + lax.axis_index("c")
    tc = wid // NBG
    bg = wid % NBG

    toks = (tok0, tok1, tok2, tok3, tok4, tok5)
    gsems = (gsem0, gsem1, gsem2, gsem3, gsem4, gsem5)
    osems = (osem0, osem1, osem2, osem3, osem4, osem5)

    def row_base(g):
        return (bg * GB + g) * T + tc * C

    pltpu.sync_copy(
        x_hbm.at[pl.ds(bg * GB, GB), pl.ds(tc * C, C)], idx_v)
    pos_copy = pltpu.async_copy(pos_hbm.at[pl.ds(tc * C, C)], pos_v, psem)

    def add_pos(tok_ref):
        def row_body(r, c2):
            for j in range(D // LANES):
                sl = pl.ds(j * LANES, LANES)
                plsc.addupdate(tok_ref.at[r, sl], pos_v[r, sl])
            return c2
        lax.fori_loop(0, C, row_body, 0, unroll=8)

    gathers = [None] * GB
    out_copies = [None] * NBUF

    def issue_gather(g):
        buf = g % NBUF
        if out_copies[buf] is not None:
            out_copies[buf].wait()
            out_copies[buf] = None
        gathers[g] = pltpu.async_copy(
            tok_hbm.at[idx_v.at[g]], toks[buf], gsems[buf])

    for p in range(DEPTH):
        issue_gather(p)
    pos_copy.wait()
    for g in range(GB):
        buf = g % NBUF
        gathers[g].wait()
        add_pos(toks[buf])
        out_copies[buf] = pltpu.async_copy(
            toks[buf], out_hbm.at[pl.ds(row_base(g), C)], osems[buf])
        if g + DEPTH < GB:
            issue_gather(g + DEPTH)

    for oc in out_copies:
        if oc is not None:
            oc.wait()


@jax.jit
def _emb_call(x2d, token_table, pos_table):
    mesh = plsc.VectorSubcoreMesh(
        core_axis_name="c", subcore_axis_name="s", num_cores=NC, num_subcores=NS
    )
    f = pl.kernel(
        _emb_body,
        out_type=jax.ShapeDtypeStruct((B * T, D), jnp.float32),
        mesh=mesh,
        scratch_types=[
            pltpu.VMEM((C, D), jnp.float32),
            pltpu.VMEM((GB, C), jnp.int32),
        ] + [pltpu.VMEM((C, D), jnp.float32) for _ in range(NBUF)]
          + [pltpu.SemaphoreType.DMA for _ in range(1 + 2 * NBUF)],
    )
    return f(x2d, token_table, pos_table)


def kernel(x, token_table, pos_table):
    out = _emb_call(x.astype(jnp.int32), token_table, pos_table)
    return out.reshape(B, T, D)


# speedup vs baseline: 1.0633x; 1.0633x over previous
import jax
import jax.numpy as jnp
from jax import lax
from jax.experimental import pallas as pl
from jax.experimental.pallas import tpu as pltpu
from jax.experimental.pallas import tpu_sc as plsc

B = 32
T = 2048
D = 128
C = 128
NC = 2
NS = 16
NW = NC * NS
NTC = T // C
NBG = NW // NTC
GB = B // NBG
LANES = 16
NBUF = 4
DEPTH = 3


def _emb_body(x_hbm, tok_hbm, pos_hbm, out_hbm,
              pos_v, idx_v, tok0, tok1, tok2, tok3,
              psem,
              gsem0, gsem1, gsem2, gsem3,
              osem0, osem1, osem2, osem3):
    wid = lax.axis_index("s") * NC + lax.axis_index("c")
    tc = wid // NBG
    bg = wid % NBG

    toks = (tok0, tok1, tok2, tok3)
    gsems = (gsem0, gsem1, gsem2, gsem3)
    osems = (osem0, osem1, osem2, osem3)

    def row_base(g):
        return (bg * GB + g) * T + tc * C

    pltpu.sync_copy(
        x_hbm.at[pl.ds(bg * GB, GB), pl.ds(tc * C, C)], idx_v)
    pos_copy = pltpu.async_copy(pos_hbm.at[pl.ds(tc * C, C)], pos_v, psem)

    def add_pos(tok_ref):
        def row_body(r, c2):
            for j in range(D // LANES):
                sl = pl.ds(j * LANES, LANES)
                plsc.addupdate(tok_ref.at[r, sl], pos_v[r, sl])
            return c2
        lax.fori_loop(0, C, row_body, 0, unroll=4)

    gathers = [None] * GB
    out_copies = [None] * NBUF

    def issue_gather(g):
        buf = g % NBUF
        if out_copies[buf] is not None:
            out_copies[buf].wait()
            out_copies[buf] = None
        gathers[g] = pltpu.async_copy(
            tok_hbm.at[idx_v.at[g]], toks[buf], gsems[buf])

    for p in range(DEPTH):
        issue_gather(p)
    pos_copy.wait()
    for g in range(GB):
        buf = g % NBUF
        gathers[g].wait()
        add_pos(toks[buf])
        out_copies[buf] = pltpu.async_copy(
            toks[buf], out_hbm.at[pl.ds(row_base(g), C)], osems[buf])
        if g + DEPTH < GB:
            issue_gather(g + DEPTH)

    for oc in out_copies:
        if oc is not None:
            oc.wait()


@jax.jit
def _emb_call(x2d, token_table, pos_table):
    mesh = plsc.VectorSubcoreMesh(
        core_axis_name="c", subcore_axis_name="s", num_cores=NC, num_subcores=NS
    )
    f = pl.kernel(
        _emb_body,
        out_type=jax.ShapeDtypeStruct((B * T, D), jnp.float32),
        mesh=mesh,
        scratch_types=[
            pltpu.VMEM((C, D), jnp.float32),
            pltpu.VMEM((GB, C), jnp.int32),
        ] + [pltpu.VMEM((C, D), jnp.float32) for _ in range(NBUF)]
          + [pltpu.SemaphoreType.DMA for _ in range(1 + 2 * NBUF)],
    )
    return f(x2d, token_table, pos_table)


def kernel(x, token_table, pos_table):
    out = _emb_call(x.astype(jnp.int32), token_table, pos_table)
    return out.reshape(B, T, D)


# speedup vs baseline: 1.0840x; 1.0195x over previous
import jax
import jax.numpy as jnp
from jax import lax
from jax.experimental import pallas as pl
from jax.experimental.pallas import tpu as pltpu
from jax.experimental.pallas import tpu_sc as plsc

B = 32
T = 2048
D = 128
C = 128
NC = 2
NS = 16
NW = NC * NS
NTC = T // C
NBG = NW // NTC
GB = B // NBG
LANES = 16
NBUF = 6
DEPTH = 5


def _emb_body(x_hbm, tok_hbm, pos_hbm, out_hbm,
              pos_v, idx_v, tok0, tok1, tok2, tok3, tok4, tok5,
              psem,
              gsem0, gsem1, gsem2, gsem3, gsem4, gsem5,
              osem0, osem1, osem2, osem3, osem4, osem5):
    wid = lax.axis_index("s") * NC + lax.axis_index("c")
    tc = wid // NBG
    bg = wid % NBG

    toks = (tok0, tok1, tok2, tok3, tok4, tok5)
    gsems = (gsem0, gsem1, gsem2, gsem3, gsem4, gsem5)
    osems = (osem0, osem1, osem2, osem3, osem4, osem5)

    def row_base(g):
        return (bg * GB + g) * T + tc * C

    pltpu.sync_copy(
        x_hbm.at[pl.ds(bg * GB, GB), pl.ds(tc * C, C)], idx_v)
    pos_copy = pltpu.async_copy(pos_hbm.at[pl.ds(tc * C, C)], pos_v, psem)

    def add_pos(tok_ref):
        def row_body(r, c2):
            for j in range(D // LANES):
                sl = pl.ds(j * LANES, LANES)
                plsc.addupdate(tok_ref.at[r, sl], pos_v[r, sl])
            return c2
        lax.fori_loop(0, C, row_body, 0, unroll=4)

    gathers = [None] * GB
    out_copies = [None] * NBUF

    def issue_gather(g):
        buf = g % NBUF
        if out_copies[buf] is not None:
            out_copies[buf].wait()
            out_copies[buf] = None
        gathers[g] = pltpu.async_copy(
            tok_hbm.at[idx_v.at[g]], toks[buf], gsems[buf])

    for p in range(DEPTH):
        issue_gather(p)
    pos_copy.wait()
    for g in range(GB):
        buf = g % NBUF
        gathers[g].wait()
        add_pos(toks[buf])
        out_copies[buf] = pltpu.async_copy(
            toks[buf], out_hbm.at[pl.ds(row_base(g), C)], osems[buf])
        if g + DEPTH < GB:
            issue_gather(g + DEPTH)

    for oc in out_copies:
        if oc is not None:
            oc.wait()


@jax.jit
def _emb_call(x2d, token_table, pos_table):
    mesh = plsc.VectorSubcoreMesh(
        core_axis_name="c", subcore_axis_name="s", num_cores=NC, num_subcores=NS
    )
    f = pl.kernel(
        _emb_body,
        out_type=jax.ShapeDtypeStruct((B * T, D), jnp.float32),
        mesh=mesh,
        scratch_types=[
            pltpu.VMEM((C, D), jnp.float32),
            pltpu.VMEM((GB, C), jnp.int32),
        ] + [pltpu.VMEM((C, D), jnp.float32) for _ in range(NBUF)]
          + [pltpu.SemaphoreType.DMA for _ in range(1 + 2 * NBUF)],
    )
    return f(x2d, token_table, pos_table)


def kernel(x, token_table, pos_table):
    out = _emb_call(x.astype(jnp.int32), token_table, pos_table)
    return out.reshape(B, T, D)
